# baseline (speedup 1.0000x reference)
import functools

import jax
import jax.numpy as jnp
from jax import lax
from jax.experimental import pallas as pl
from jax.experimental.pallas import tpu as pltpu
from jax.experimental.pallas import tpu_sc as plsc

_RATIOS = (0.5, 1.0, 2.0)
_SCALES = (8, 16, 32)
_IMAGE_SIZE = 1920
_NMS_PRE = 3000
_NMS_POST = 300
_THRESHOLD = 0.5
_NEG = -1e30
_LANES = 128

_SC_SUBCORES = 16

_COMPACT = 3328
_N_ROWS = 1024
_ROWS_PER_SUB = _N_ROWS // _SC_SUBCORES


def _make_anchors(feat_stride, size):
    ratios = jnp.asarray(_RATIOS, jnp.float32)
    scales = jnp.asarray(_SCALES, jnp.float32)
    base = feat_stride * scales
    ws = (base[None, :] * jnp.sqrt(1.0 / ratios)[:, None]).reshape(-1)
    hs = (base[None, :] * jnp.sqrt(ratios)[:, None]).reshape(-1)
    ctr = (jnp.arange(size, dtype=jnp.float32) + 0.5) * feat_stride
    cy = ctr[:, None, None]
    cx = ctr[None, :, None]
    x1, y1, x2, y2 = jnp.broadcast_arrays(
        cx - ws / 2, cy - hs / 2, cx + ws / 2, cy + hs / 2)
    return jnp.stack([x1, y1, x2, y2], axis=-1)


def _scramble(arr, K, H, W):
    lead = arr.shape[:-4]
    a = arr.reshape(lead + (K, 4, H, W))
    perm = tuple(range(len(lead))) + tuple(
        len(lead) + p for p in (2, 3, 0, 1))
    return jnp.transpose(a, perm).reshape(lead + (H * W * K, 4))


def _decode_select_body(k_top, trash,
                        s_ref, dx1, dy1, dx2, dy2, ax1, ay1, ax2, ay2,
                        os_ref, ox1, oy1, ox2, oy2, od_ref):
    hi = jnp.float32(_IMAGE_SIZE)
    ox1[0] = jnp.clip(dx1[0] + ax1[...], 0.0, hi)
    oy1[0] = jnp.clip(dy1[0] + ay1[...], 0.0, hi)
    ox2[0] = jnp.clip(dx2[0] + ax2[...], 0.0, hi)
    oy2[0] = jnp.clip(dy2[0] + ay2[...], 0.0, hi)

    s = s_ref[0]
    u = lax.bitcast_convert_type(s, jnp.int32)
    m = u ^ (jnp.right_shift(u, 31) & jnp.int32(0x7FFFFFFF))

    def count_ge(t):
        return jnp.sum((m >= t).astype(jnp.int32))

    p0 = jnp.where(count_ge(jnp.int32(0)) >= k_top,
                   jnp.int32(0), jnp.int32(-2**31))

    def bit_step(i, p):
        b = jnp.int32(30) - i
        t = p + jnp.left_shift(jnp.int32(1), b)
        return jnp.where(count_ge(t) >= k_top, t, p)

    t_key = lax.fori_loop(0, 31, bit_step, p0)

    c_gt = jnp.sum((m > t_key).astype(jnp.int32))
    need = jnp.int32(k_top) - c_gt
    eq = m == t_key
    fidx = (lax.broadcasted_iota(jnp.int32, m.shape, 0) * _LANES
            + lax.broadcasted_iota(jnp.int32, m.shape, 1))

    def idx_step(i, r):
        b = jnp.int32(17) - i
        t = r + jnp.left_shift(jnp.int32(1), b)
        cl = jnp.sum((eq & (fidx < t)).astype(jnp.int32))
        return jnp.where(cl < need, t, r)

    e = lax.fori_loop(0, 18, idx_step, jnp.int32(0))
    sel = (m > t_key) | (eq & (fidx <= e))
    os_ref[0] = jnp.where(sel, s, jnp.float32(_NEG))

    self32 = sel.astype(jnp.float32)
    slt = (lax.broadcasted_iota(jnp.int32, (_LANES, _LANES), 0)
           < lax.broadcasted_iota(jnp.int32, (_LANES, _LANES), 1)
           ).astype(jnp.float32)
    excl_in_row = jnp.dot(self32, slt,
                          preferred_element_type=jnp.float32)
    row_tot = jnp.sum(self32, axis=1, keepdims=True)
    row_lt = (lax.broadcasted_iota(jnp.int32, (_N_ROWS, _N_ROWS), 1)
              < lax.broadcasted_iota(jnp.int32, (_N_ROWS, _N_ROWS), 0)
              ).astype(jnp.float32)
    row_excl = jnp.dot(row_lt, row_tot,
                       preferred_element_type=jnp.float32)
    rank = (excl_in_row + row_excl).astype(jnp.int32)

    img = pl.program_id(0)
    dst = jnp.where(sel, img * _COMPACT + rank,
                    jnp.int32(trash) + (fidx & 7))
    od_ref[0] = dst


def _sc_scatter_body(s_in, x1_in, y1_in, x2_in, y2_in, d_in,
                     out_s, out_x1, out_y1, out_x2, out_y2,
                     vb_s, vb_x1, vb_y1, vb_x2, vb_y2, vb_d,
                     fillbuf, sems):
    img = lax.axis_index("c")
    sid = lax.axis_index("s")
    chunk = _ROWS_PER_SUB * _LANES
    base_in = pl.multiple_of((img * _SC_SUBCORES + sid) * chunk, 8)

    vbufs = (vb_s, vb_x1, vb_y1, vb_x2, vb_y2)
    outs = (out_s, out_x1, out_y1, out_x2, out_y2)

    for src, vb in zip((s_in, x1_in, y1_in, x2_in, y2_in), vbufs):
        pltpu.sync_copy(src.at[pl.ds(base_in, chunk)], vb)
    pltpu.sync_copy(d_in.at[pl.ds(base_in, chunk)], vb_d)

    copies = [
        pltpu.async_copy(vb, out.at[vb_d], sem)
        for (vb, out, sem) in zip(vbufs, outs, sems)
    ]
    for c in copies:
        c.wait()

    tail = _COMPACT - _NMS_PRE

    @pl.when(sid == 0)
    def _():
        for t in range(tail // 16):
            fillbuf[pl.ds(t * 16, 16)] = jnp.full(
                (16,), _NEG, jnp.float32)
        tbase = pl.multiple_of(img * _COMPACT + _NMS_PRE, 8)
        pltpu.sync_copy(fillbuf.at[pl.ds(0, tail)],
                        out_s.at[pl.ds(tbase, tail)])
        for t in range(tail // 16):
            fillbuf[pl.ds(t * 16, 16)] = jnp.zeros((16,), jnp.float32)
        for out in outs[1:]:
            pltpu.sync_copy(fillbuf.at[pl.ds(0, tail)],
                            out.at[pl.ds(tbase, tail)])


def _nms_body(n_post, s_ref, x1_ref, y1_ref, x2_ref, y2_ref,
              out_ref, areas, snms):
    snms[...] = s_ref[0]
    areas[...] = (x2_ref[0] - x1_ref[0]) * (y2_ref[0] - y1_ref[0])
    fidx = (lax.broadcasted_iota(jnp.int32, snms.shape, 0) * _LANES
            + lax.broadcasted_iota(jnp.int32, snms.shape, 1))
    li = lax.broadcasted_iota(jnp.int32, (1, _LANES), 1)

    def nms_step(t, carry):
        sarr = snms[...]
        mval = jnp.max(sarr)
        fi = jnp.min(jnp.where(sarr == mval, fidx, jnp.int32(2**30)))
        valid = mval > jnp.float32(_NEG)
        row = fi // _LANES
        lane = fi % _LANES

        def pick(ref):
            rowv = ref[0, pl.ds(row, 1), :]
            return jnp.sum(jnp.where(li == lane, rowv, 0.0))

        bx1 = pick(x1_ref)
        by1 = pick(y1_ref)
        bx2 = pick(x2_ref)
        by2 = pick(y2_ref)
        rowa = areas[pl.ds(row, 1), :]
        ba = jnp.sum(jnp.where(li == lane, rowa, 0.0))

        xx1 = jnp.maximum(x1_ref[0], bx1)
        yy1 = jnp.maximum(y1_ref[0], by1)
        xx2 = jnp.minimum(x2_ref[0], bx2)
        yy2 = jnp.minimum(y2_ref[0], by2)
        inter = (jnp.maximum(xx2 - xx1, 0.0) * jnp.maximum(yy2 - yy1, 0.0))
        iou = inter / (areas[...] + ba - inter + jnp.float32(1e-9))
        kill = (iou > jnp.float32(_THRESHOLD)) | (fidx == fi)
        snms[...] = jnp.where(kill, jnp.float32(_NEG), sarr)

        z = jnp.float32(0.0)
        kx1 = jnp.where(valid, bx1, z)
        ky1 = jnp.where(valid, by1, z)
        kx2 = jnp.where(valid, bx2, z)
        ky2 = jnp.where(valid, by2, z)
        ks = jnp.where(valid, mval, z)
        vals = jnp.where(
            li == 0, kx1,
            jnp.where(li == 1, ky1,
                      jnp.where(li == 2, kx2,
                                jnp.where(li == 3, ky2,
                                          jnp.where(li == 4, ks, z)))))
        out_ref[0, pl.ds(t, 1), :] = vals
        return carry

    lax.fori_loop(0, n_post, nms_step, jnp.int32(0))


def kernel(cls_scores, bbox_deltas, device):
    N, C, H, W = cls_scores.shape
    K = C
    A = C * H * W
    feat_stride = round(_IMAGE_SIZE / float(W))

    anchors = _make_anchors(feat_stride, W)
    anchors_flat = _scramble(anchors, K, H, W)

    deltas = jnp.transpose(bbox_deltas, (0, 2, 3, 1)).reshape(N, H, W, K, 4)
    deltas_flat = _scramble(deltas, K, H, W)

    scores_flat = cls_scores.reshape(N, A)

    tot = _N_ROWS * _LANES
    pad = tot - A
    trash = N * _COMPACT

    s_pad = jnp.concatenate(
        [scores_flat, jnp.full((N, pad), -jnp.inf, jnp.float32)],
        axis=1).reshape(N, _N_ROWS, _LANES)

    def pad_plane(x):
        lead = x.shape[:-1]
        z = jnp.zeros(lead + (pad,), jnp.float32)
        return jnp.concatenate([x, z], axis=-1).reshape(
            lead + (_N_ROWS, _LANES))

    d_planes = [pad_plane(deltas_flat[..., c]) for c in range(4)]
    a_planes = [pad_plane(anchors_flat[..., c]) for c in range(4)]

    img_spec = pl.BlockSpec((1, _N_ROWS, _LANES), lambda i: (i, 0, 0))
    const_spec = pl.BlockSpec((_N_ROWS, _LANES), lambda i: (0, 0))
    plane_sds = jax.ShapeDtypeStruct((N, _N_ROWS, _LANES), jnp.float32)
    idx_sds = jax.ShapeDtypeStruct((N, _N_ROWS, _LANES), jnp.int32)

    sel_body = functools.partial(_decode_select_body, _NMS_PRE, trash)
    sm, px1, py1, px2, py2, pdst = pl.pallas_call(
        sel_body,
        grid=(N,),
        in_specs=[img_spec] + [img_spec] * 4 + [const_spec] * 4,
        out_specs=[img_spec] * 6,
        out_shape=[plane_sds] * 5 + [idx_sds],
        compiler_params=pltpu.CompilerParams(
            dimension_semantics=("parallel",)),
    )(s_pad, *d_planes, *a_planes)

    mesh = plsc.VectorSubcoreMesh(core_axis_name="c", subcore_axis_name="s")
    sc_out = jax.ShapeDtypeStruct((N * _COMPACT + 8,), jnp.float32)
    r2 = (N * _N_ROWS * _LANES,)
    sc_fn = pl.kernel(
        _sc_scatter_body,
        mesh=mesh,
        out_type=[sc_out] * 5,
        scratch_types=(
            [pltpu.VMEM((_ROWS_PER_SUB * _LANES,), jnp.float32)] * 5
            + [pltpu.VMEM((_ROWS_PER_SUB * _LANES,), jnp.int32),
               pltpu.VMEM((_COMPACT - _NMS_PRE,), jnp.float32),
               [pltpu.SemaphoreType.DMA] * 5]),
    )
    cs, cx1, cy1, cx2, cy2 = sc_fn(
        sm.reshape(r2), px1.reshape(r2), py1.reshape(r2),
        px2.reshape(r2), py2.reshape(r2), pdst.reshape(r2))

    c_rows = _COMPACT // _LANES
    cshape = (N, c_rows, _LANES)
    cimg_spec = pl.BlockSpec((1, c_rows, _LANES), lambda i: (i, 0, 0))

    def crop(x):
        return x[:N * _COMPACT].reshape(cshape)

    out_rows = _NMS_POST + (-_NMS_POST) % 8
    out = pl.pallas_call(
        functools.partial(_nms_body, _NMS_POST),
        grid=(N,),
        in_specs=[cimg_spec] * 5,
        out_specs=pl.BlockSpec((1, out_rows, _LANES), lambda i: (i, 0, 0)),
        out_shape=jax.ShapeDtypeStruct((N, out_rows, _LANES), jnp.float32),
        scratch_shapes=[pltpu.VMEM((c_rows, _LANES), jnp.float32)
                        for _ in range(2)],
        compiler_params=pltpu.CompilerParams(
            dimension_semantics=("parallel",)),
    )(crop(cs), crop(cx1), crop(cy1), crop(cx2), crop(cy2))

    boxes = out[:, :_NMS_POST, 0:4]
    last_scores = out[N - 1, :_NMS_POST, 4]
    scores_col = jnp.broadcast_to(last_scores[None, :], (N, _NMS_POST))
    return jnp.concatenate([scores_col[..., None], boxes], axis=-1)


# speedup vs baseline: 80.5706x; 80.5706x over previous
import functools

import jax
import jax.numpy as jnp
from jax import lax
from jax.experimental import pallas as pl
from jax.experimental.pallas import tpu as pltpu

_RATIOS = (0.5, 1.0, 2.0)
_SCALES = (8, 16, 32)
_IMAGE_SIZE = 1920
_NMS_PRE = 3000
_NMS_POST = 300
_THRESHOLD = 0.5
_NEG = -1e30
_LANES = 128


def _make_anchors(feat_stride, size):
    ratios = jnp.asarray(_RATIOS, jnp.float32)
    scales = jnp.asarray(_SCALES, jnp.float32)
    base = feat_stride * scales
    ws = (base[None, :] * jnp.sqrt(1.0 / ratios)[:, None]).reshape(-1)
    hs = (base[None, :] * jnp.sqrt(ratios)[:, None]).reshape(-1)
    ctr = (jnp.arange(size, dtype=jnp.float32) + 0.5) * feat_stride
    cy = ctr[:, None, None]
    cx = ctr[None, :, None]
    x1, y1, x2, y2 = jnp.broadcast_arrays(
        cx - ws / 2, cy - hs / 2, cx + ws / 2, cy + hs / 2)
    return jnp.stack([x1, y1, x2, y2], axis=-1)


def _scramble(arr, K, H, W):
    lead = arr.shape[:-4]
    a = arr.reshape(lead + (K, 4, H, W))
    perm = tuple(range(len(lead))) + tuple(
        len(lead) + p for p in (2, 3, 0, 1))
    return jnp.transpose(a, perm).reshape(lead + (H * W * K, 4))


def _nms_body(n_rows, n_post, k_top,
              s_ref, dx1, dy1, dx2, dy2, ax1, ay1, ax2, ay2,
              out_ref, rx1, ry1, rx2, ry2, areas, snms):
    hi = jnp.float32(_IMAGE_SIZE)
    rx1[...] = jnp.clip(dx1[0] + ax1[...], 0.0, hi)
    ry1[...] = jnp.clip(dy1[0] + ay1[...], 0.0, hi)
    rx2[...] = jnp.clip(dx2[0] + ax2[...], 0.0, hi)
    ry2[...] = jnp.clip(dy2[0] + ay2[...], 0.0, hi)
    areas[...] = (rx2[...] - rx1[...]) * (ry2[...] - ry1[...])

    s = s_ref[0]
    u = lax.bitcast_convert_type(s, jnp.int32)
    m = u ^ (jnp.right_shift(u, 31) & jnp.int32(0x7FFFFFFF))

    def count_ge(t):
        return jnp.sum((m >= t).astype(jnp.int32))

    p0 = jnp.where(count_ge(jnp.int32(0)) >= k_top,
                   jnp.int32(0), jnp.int32(-2**31))

    def bit_step(i, p):
        b = jnp.int32(30) - i
        t = p + jnp.left_shift(jnp.int32(1), b)
        return jnp.where(count_ge(t) >= k_top, t, p)

    t_key = lax.fori_loop(0, 31, bit_step, p0)

    c_gt = jnp.sum((m > t_key).astype(jnp.int32))
    need = jnp.int32(k_top) - c_gt
    eq = m == t_key
    fidx = (lax.broadcasted_iota(jnp.int32, m.shape, 0) * _LANES
            + lax.broadcasted_iota(jnp.int32, m.shape, 1))

    def idx_step(i, r):
        b = jnp.int32(17) - i
        t = r + jnp.left_shift(jnp.int32(1), b)
        cl = jnp.sum((eq & (fidx < t)).astype(jnp.int32))
        return jnp.where(cl < need, t, r)

    e = lax.fori_loop(0, 18, idx_step, jnp.int32(0))
    sel = (m > t_key) | (eq & (fidx <= e))
    snms[...] = jnp.where(sel, s, jnp.float32(_NEG))

    li = lax.broadcasted_iota(jnp.int32, (1, _LANES), 1)

    def nms_step(t, carry):
        sarr = snms[...]
        mval = jnp.max(sarr)
        fi = jnp.min(jnp.where(sarr == mval, fidx, jnp.int32(2**30)))
        valid = mval > jnp.float32(_NEG)
        row = fi // _LANES
        lane = fi % _LANES

        def pick(ref):
            rowv = ref[pl.ds(row, 1), :]
            return jnp.sum(jnp.where(li == lane, rowv, 0.0))

        bx1 = pick(rx1)
        by1 = pick(ry1)
        bx2 = pick(rx2)
        by2 = pick(ry2)
        ba = pick(areas)

        xx1 = jnp.maximum(rx1[...], bx1)
        yy1 = jnp.maximum(ry1[...], by1)
        xx2 = jnp.minimum(rx2[...], bx2)
        yy2 = jnp.minimum(ry2[...], by2)
        inter = (jnp.maximum(xx2 - xx1, 0.0) * jnp.maximum(yy2 - yy1, 0.0))
        iou = inter / (areas[...] + ba - inter + jnp.float32(1e-9))
        kill = (iou > jnp.float32(_THRESHOLD)) | (fidx == fi)
        snms[...] = jnp.where(kill, jnp.float32(_NEG), sarr)

        z = jnp.float32(0.0)
        kx1 = jnp.where(valid, bx1, z)
        ky1 = jnp.where(valid, by1, z)
        kx2 = jnp.where(valid, bx2, z)
        ky2 = jnp.where(valid, by2, z)
        ks = jnp.where(valid, mval, z)
        vals = jnp.where(
            li == 0, kx1,
            jnp.where(li == 1, ky1,
                      jnp.where(li == 2, kx2,
                                jnp.where(li == 3, ky2,
                                          jnp.where(li == 4, ks, z)))))
        out_ref[0, pl.ds(t, 1), :] = vals
        return carry

    lax.fori_loop(0, n_post, nms_step, jnp.int32(0))


def kernel(cls_scores, bbox_deltas, device):
    N, C, H, W = cls_scores.shape
    K = C
    A = C * H * W
    feat_stride = round(_IMAGE_SIZE / float(W))

    anchors = _make_anchors(feat_stride, W)
    anchors_flat = _scramble(anchors, K, H, W)

    deltas = jnp.transpose(bbox_deltas, (0, 2, 3, 1)).reshape(N, H, W, K, 4)
    deltas_flat = _scramble(deltas, K, H, W)

    scores_flat = cls_scores.reshape(N, A)

    n_rows = -(-A // _LANES)
    n_rows += (-n_rows) % 8
    tot = n_rows * _LANES
    pad = tot - A

    s_pad = jnp.concatenate(
        [scores_flat, jnp.full((N, pad), -jnp.inf, jnp.float32)],
        axis=1).reshape(N, n_rows, _LANES)

    def pad_plane(x):
        lead = x.shape[:-1]
        z = jnp.zeros(lead + (pad,), jnp.float32)
        return jnp.concatenate([x, z], axis=-1).reshape(
            lead + (n_rows, _LANES))

    d_planes = [pad_plane(deltas_flat[..., c]) for c in range(4)]
    a_planes = [pad_plane(anchors_flat[..., c]) for c in range(4)]

    out_rows = _NMS_POST + (-_NMS_POST) % 8

    img_spec = pl.BlockSpec((1, n_rows, _LANES), lambda i: (i, 0, 0))
    const_spec = pl.BlockSpec((n_rows, _LANES), lambda i: (0, 0))

    body = functools.partial(_nms_body, n_rows, _NMS_POST, _NMS_PRE)
    out = pl.pallas_call(
        body,
        grid=(N,),
        in_specs=[img_spec] + [img_spec] * 4 + [const_spec] * 4,
        out_specs=pl.BlockSpec((1, out_rows, _LANES), lambda i: (i, 0, 0)),
        out_shape=jax.ShapeDtypeStruct((N, out_rows, _LANES), jnp.float32),
        scratch_shapes=[pltpu.VMEM((n_rows, _LANES), jnp.float32)
                        for _ in range(6)],
        compiler_params=pltpu.CompilerParams(
            dimension_semantics=("parallel",)),
    )(s_pad, *d_planes, *a_planes)

    boxes = out[:, :_NMS_POST, 0:4]
    last_scores = out[N - 1, :_NMS_POST, 4]
    scores_col = jnp.broadcast_to(last_scores[None, :], (N, _NMS_POST))
    return jnp.concatenate([scores_col[..., None], boxes], axis=-1)
